# baseline (speedup 1.0000x reference)
import functools

import jax
import jax.numpy as jnp
import numpy as np
from jax import lax
from jax.experimental import pallas as pl
from jax.experimental.pallas import tpu as pltpu
from jax.experimental.pallas import tpu_sc as plsc

N_NODES = 10000
N_EDGES = 320000
D_IN = 128
D_HID = 256
D_OUT = 256

NC = 2
NS = 16
NW = NC * NS
BLK = 128
CHK = 8
NCHK1 = 10
NCHK2 = 20
NBLK1 = CHK * NCHK1
NBLK2 = CHK * NCHK2
E_PAD = NW * NBLK1 * BLK
RPT = 632
ACC_N = NS * RPT

_NPAD = E_PAD - N_EDGES
_SRC_TMPL = jnp.asarray(np.arange(E_PAD, dtype=np.int32) % N_NODES)
_DST_TMPL = jnp.asarray(
    N_NODES + np.arange(E_PAD, dtype=np.int32) % (ACC_N - N_NODES))

_sc_mesh = plsc.VectorSubcoreMesh(core_axis_name="c", subcore_axis_name="s")


def _segsum_loop(tab_hbm, idx_load, idx_wait, src_v, dst_v, rows_v, acc_sh,
                 sem, nchk, per_block=None):
    pltpu.async_copy(tab_hbm.at[src_v.at[0, 0]], rows_v.at[0], sem)

    def chunk(g, carry):
        par = g % 2

        @pl.when(g < nchk - 1)
        def _():
            idx_load(g + 1)

        for b in range(CHK):
            if b < CHK - 1:
                pltpu.async_copy(tab_hbm.at[src_v.at[par, b + 1]],
                                 rows_v.at[(b + 1) % 2], sem)
            else:
                @pl.when(g < nchk - 1)
                def _():
                    idx_wait()
                    pltpu.async_copy(tab_hbm.at[src_v.at[(g + 1) % 2, 0]],
                                     rows_v.at[0], sem)
            if per_block is not None:
                per_block(par, b)
            pltpu.make_async_copy(tab_hbm.at[src_v.at[par, b]],
                                  rows_v.at[b % 2], sem).wait()
            pltpu.sync_copy(rows_v.at[b % 2], acc_sh.at[dst_v.at[par, b]],
                            add=True)
        return carry

    lax.fori_loop(0, nchk, chunk, 0)


@functools.partial(
    pl.kernel,
    mesh=_sc_mesh,
    out_type=[
        jax.ShapeDtypeStruct((NC, ACC_N, D_IN), jnp.float32),
        jax.ShapeDtypeStruct((NC * ACC_N,), jnp.float32),
    ],
    scratch_types=[
        pltpu.VMEM_SHARED((ACC_N, D_IN), jnp.float32),
        pltpu.VMEM_SHARED((ACC_N,), jnp.float32),
        pltpu.VMEM((2, CHK, BLK), jnp.int32),
        pltpu.VMEM((2, CHK, BLK), jnp.int32),
        pltpu.VMEM((2, BLK, D_IN), jnp.float32),
        pltpu.VMEM((BLK,), jnp.float32),
        pltpu.VMEM((RPT,), jnp.float32),
        pltpu.SemaphoreType.DMA,
        pltpu.SemaphoreType.DMA,
    ],
)
def _sc_segsum1(x_hbm, src_hbm, dst_hbm, z2_hbm, z1_hbm, ones_hbm,
                sum_out, deg_out, acc_sh, deg_sh, src_v, dst_v, rows_v,
                ones_v, deg_v, sem, isem):
    c = lax.axis_index("c")
    s = lax.axis_index("s")
    wid = c * NS + s
    pltpu.sync_copy(ones_hbm, ones_v)
    pltpu.sync_copy(src_hbm.at[wid, pl.ds(0, CHK)], src_v.at[0])
    pltpu.sync_copy(dst_hbm.at[wid, pl.ds(0, CHK)], dst_v.at[0])
    pltpu.sync_copy(z2_hbm.at[pl.ds(s * RPT, RPT)], acc_sh.at[pl.ds(s * RPT, RPT)])
    pltpu.sync_copy(z1_hbm.at[pl.ds(s * RPT, RPT)], deg_v)
    pltpu.sync_copy(deg_v, deg_sh.at[pl.ds(s * RPT, RPT)])
    plsc.subcore_barrier()

    def idx_load(g):
        pltpu.async_copy(src_hbm.at[wid, pl.ds(g * CHK, CHK)],
                         src_v.at[g % 2], isem)
        pltpu.async_copy(dst_hbm.at[wid, pl.ds(g * CHK, CHK)],
                         dst_v.at[g % 2], isem)

    def idx_wait():
        pltpu.make_async_copy(src_hbm.at[wid, pl.ds(0, CHK)], src_v.at[0],
                              isem).wait()
        pltpu.make_async_copy(dst_hbm.at[wid, pl.ds(0, CHK)], dst_v.at[0],
                              isem).wait()

    def per_block(par, b):
        pltpu.sync_copy(ones_v, deg_sh.at[dst_v.at[par, b]], add=True)

    _segsum_loop(x_hbm, idx_load, idx_wait, src_v, dst_v, rows_v, acc_sh,
                 sem, NCHK1, per_block)
    plsc.subcore_barrier()
    pltpu.sync_copy(acc_sh.at[pl.ds(s * RPT, RPT)], sum_out.at[c, pl.ds(s * RPT, RPT)])
    pltpu.sync_copy(deg_sh.at[pl.ds(s * RPT, RPT)], deg_v)
    pltpu.sync_copy(deg_v, deg_out.at[pl.ds(c * ACC_N + s * RPT, RPT)])


@functools.partial(
    pl.kernel,
    mesh=_sc_mesh,
    out_type=jax.ShapeDtypeStruct((NC, ACC_N, D_IN), jnp.float32),
    scratch_types=[
        pltpu.VMEM_SHARED((ACC_N, D_IN), jnp.float32),
        pltpu.VMEM((2, CHK, BLK), jnp.int32),
        pltpu.VMEM((2, CHK, BLK), jnp.int32),
        pltpu.VMEM((2, BLK, D_IN), jnp.float32),
        pltpu.SemaphoreType.DMA,
        pltpu.SemaphoreType.DMA,
    ],
)
def _sc_segsum2(h2_hbm, src2_hbm, dst2_hbm, z2_hbm,
                sum_out, acc_sh, src_v, dst_v, rows_v, sem, isem):
    c = lax.axis_index("c")
    s = lax.axis_index("s")
    pltpu.sync_copy(src2_hbm.at[c, s, pl.ds(0, CHK)], src_v.at[0])
    pltpu.sync_copy(dst2_hbm.at[s, pl.ds(0, CHK)], dst_v.at[0])
    pltpu.sync_copy(z2_hbm.at[pl.ds(s * RPT, RPT)],
                    acc_sh.at[pl.ds(s * RPT, RPT)])
    plsc.subcore_barrier()

    def idx_load(g):
        pltpu.async_copy(src2_hbm.at[c, s, pl.ds(g * CHK, CHK)],
                         src_v.at[g % 2], isem)
        pltpu.async_copy(dst2_hbm.at[s, pl.ds(g * CHK, CHK)],
                         dst_v.at[g % 2], isem)

    def idx_wait():
        pltpu.make_async_copy(src2_hbm.at[c, s, pl.ds(0, CHK)], src_v.at[0],
                              isem).wait()
        pltpu.make_async_copy(dst2_hbm.at[s, pl.ds(0, CHK)], dst_v.at[0],
                              isem).wait()

    _segsum_loop(h2_hbm, idx_load, idx_wait, src_v, dst_v, rows_v, acc_sh,
                 sem, NCHK2)
    plsc.subcore_barrier()
    pltpu.sync_copy(acc_sh.at[pl.ds(s * RPT, RPT)],
                    sum_out.at[c, pl.ds(s * RPT, RPT)])


_DN = (((1,), (1,)), ((), ()))


def _dense_xr_body(x_ref, wr_ref, b_ref, out_ref):
    out_ref[...] = lax.dot_general(
        x_ref[...], wr_ref[...], _DN,
        preferred_element_type=jnp.float32) + b_ref[...]


def _dense1_body(parts_ref, degp_ref, xr_ref, wl_ref, h_ref):
    summed = parts_ref[0] + parts_ref[1]
    deg = jnp.maximum(degp_ref[0] + degp_ref[1], 1.0)
    agg = summed * (1.0 / deg)
    z = lax.dot_general(agg, wl_ref[...], _DN,
                        preferred_element_type=jnp.float32) + xr_ref[...]
    h = jnp.maximum(z, 0.0)
    h_ref[0] = h[:, :D_IN]
    h_ref[1] = h[:, D_IN:]


def _dense_hr_body(hlo_ref, hhi_ref, wr_ref, b_ref, out_ref):
    out_ref[...] = (
        lax.dot_general(hlo_ref[0], wr_ref[:, :D_IN], _DN,
                        preferred_element_type=jnp.float32)
        + lax.dot_general(hhi_ref[0], wr_ref[:, D_IN:], _DN,
                          preferred_element_type=jnp.float32)
        + b_ref[...])


def _dense2_body(parts_ref, degp_ref, hr_ref, wl_ref, out_ref):
    rdeg = 1.0 / jnp.maximum(degp_ref[0] + degp_ref[1], 1.0)
    out_ref[...] = (
        lax.dot_general(parts_ref[0] * rdeg, wl_ref[:, :D_IN], _DN,
                        preferred_element_type=jnp.float32)
        + lax.dot_general(parts_ref[1] * rdeg, wl_ref[:, D_IN:], _DN,
                          preferred_element_type=jnp.float32)
        + hr_ref[...])


_BR = 2000
_NRB = N_NODES // _BR


def _full(shape):
    n = len(shape)
    return pl.BlockSpec(shape, lambda *a: (0,) * n)


def kernel(x, edge_index, W1_l, b1, W1_r, W2_l, b2, W2_r):
    src = edge_index[0].astype(jnp.int32)
    dst = edge_index[1].astype(jnp.int32)
    src_p = lax.dynamic_update_slice(_SRC_TMPL, src, (0,))
    dst_p = lax.dynamic_update_slice(_DST_TMPL, dst, (0,))
    src3 = src_p.reshape(NW, NBLK1, BLK)
    dst3 = dst_p.reshape(NW, NBLK1, BLK)
    src2_3 = jnp.stack([src_p, src_p + N_NODES]).reshape(NC, NS, NBLK2, BLK)
    dst2_3 = dst_p.reshape(NS, NBLK2, BLK)
    z2 = jnp.zeros((ACC_N, D_IN), jnp.float32)
    z1 = jnp.zeros((ACC_N,), jnp.float32)
    ones = jnp.ones((BLK,), jnp.float32)

    grid = (_NRB,)
    row_spec = pl.BlockSpec((_BR, D_IN), lambda i: (i, 0))
    hid_spec = pl.BlockSpec((_BR, D_HID), lambda i: (i, 0))
    parts_spec = pl.BlockSpec((NC, _BR, D_IN), lambda i: (0, i, 0))
    deg_spec = pl.BlockSpec((NC, _BR, 1), lambda i: (0, i, 0))

    sum1, deg1 = _sc_segsum1(x, src3, dst3, z2, z1, ones)
    xr = pl.pallas_call(
        _dense_xr_body,
        grid=grid,
        in_specs=[row_spec, _full((D_HID, D_IN)), _full((1, D_HID))],
        out_specs=hid_spec,
        out_shape=jax.ShapeDtypeStruct((N_NODES, D_HID), jnp.float32),
    )(x, W1_r, b1.reshape(1, D_HID))

    deg3 = deg1.reshape(NC, ACC_N, 1)
    h2 = pl.pallas_call(
        _dense1_body,
        grid=grid,
        in_specs=[parts_spec, deg_spec, hid_spec, _full((D_HID, D_IN))],
        out_specs=pl.BlockSpec((NC, _BR, D_IN), lambda i: (0, i, 0)),
        out_shape=jax.ShapeDtypeStruct((NC, N_NODES, D_IN), jnp.float32),
    )(sum1, deg3, xr, W1_l)

    sum2 = _sc_segsum2(h2.reshape(2 * N_NODES, D_IN), src2_3, dst2_3, z2)
    hr = pl.pallas_call(
        _dense_hr_body,
        grid=grid,
        in_specs=[
            pl.BlockSpec((1, _BR, D_IN), lambda i: (0, i, 0)),
            pl.BlockSpec((1, _BR, D_IN), lambda i: (1, i, 0)),
            _full((D_OUT, D_HID)),
            _full((1, D_OUT)),
        ],
        out_specs=pl.BlockSpec((_BR, D_OUT), lambda i: (i, 0)),
        out_shape=jax.ShapeDtypeStruct((N_NODES, D_OUT), jnp.float32),
    )(h2, h2, W2_r, b2.reshape(1, D_OUT))

    out = pl.pallas_call(
        _dense2_body,
        grid=grid,
        in_specs=[parts_spec, deg_spec,
                  pl.BlockSpec((_BR, D_OUT), lambda i: (i, 0)),
                  _full((D_OUT, D_HID))],
        out_specs=pl.BlockSpec((_BR, D_OUT), lambda i: (i, 0)),
        out_shape=jax.ShapeDtypeStruct((N_NODES, D_OUT), jnp.float32),
    )(sum2, deg3, hr, W2_l)
    return out


# speedup vs baseline: 1.0956x; 1.0956x over previous
import functools

import jax
import jax.numpy as jnp
from jax import lax
from jax.experimental import pallas as pl
from jax.experimental.pallas import tpu as pltpu
from jax.experimental.pallas import tpu_sc as plsc

N_NODES = 10000
N_EDGES = 320000
D_IN = 128
D_HID = 256
D_OUT = 256

NC = 2
NS = 16
NW = NC * NS
BLK = 128
CHK = 8
NCHK1 = 10
NCHK2 = 20
NBLK1 = CHK * NCHK1
NBLK2 = CHK * NCHK2
E_PAD = NW * NBLK1 * BLK
RPT = 632
ACC_N = NS * RPT

NB = NW * NBLK1
DATA_ROWS = N_EDGES // BLK
_PBR = 496

_sc_mesh = plsc.VectorSubcoreMesh(core_axis_name="c", subcore_axis_name="s")


def _prep_body(eidx_ref, s_ref, d_ref):
    i = pl.program_id(0)
    rows = lax.broadcasted_iota(jnp.int32, (_PBR, BLK), 0) + i * _PBR
    lanes = lax.broadcasted_iota(jnp.int32, (_PBR, BLK), 1)
    flat = rows * BLK + lanes
    mask = rows < DATA_ROWS
    s_ref[...] = jnp.where(mask, eidx_ref[0].reshape(_PBR, BLK),
                           flat % N_NODES)
    d_ref[...] = jnp.where(mask, eidx_ref[1].reshape(_PBR, BLK),
                           N_NODES + flat % (ACC_N - N_NODES))


def _prep2_body(s_ref, o_ref):
    o_ref[0] = s_ref[...]
    o_ref[1] = s_ref[...] + N_NODES


def _segsum_loop(tab_hbm, idx_load, idx_wait, src_v, dst_v, rows_v, acc_sh,
                 sem, nchk, per_block=None):
    pltpu.async_copy(tab_hbm.at[src_v.at[0, 0]], rows_v.at[0], sem)

    def chunk(g, carry):
        par = g % 2

        @pl.when(g < nchk - 1)
        def _():
            idx_load(g + 1)

        for b in range(CHK):
            if b < CHK - 1:
                pltpu.async_copy(tab_hbm.at[src_v.at[par, b + 1]],
                                 rows_v.at[(b + 1) % 2], sem)
            else:
                @pl.when(g < nchk - 1)
                def _():
                    idx_wait()
                    pltpu.async_copy(tab_hbm.at[src_v.at[(g + 1) % 2, 0]],
                                     rows_v.at[0], sem)
            if per_block is not None:
                per_block(par, b)
            pltpu.make_async_copy(tab_hbm.at[src_v.at[par, b]],
                                  rows_v.at[b % 2], sem).wait()
            pltpu.sync_copy(rows_v.at[b % 2], acc_sh.at[dst_v.at[par, b]],
                            add=True)
        return carry

    lax.fori_loop(0, nchk, chunk, 0)


@functools.partial(
    pl.kernel,
    mesh=_sc_mesh,
    out_type=[
        jax.ShapeDtypeStruct((NC, ACC_N, D_IN), jnp.float32),
        jax.ShapeDtypeStruct((NC * ACC_N,), jnp.float32),
    ],
    scratch_types=[
        pltpu.VMEM_SHARED((ACC_N, D_IN), jnp.float32),
        pltpu.VMEM_SHARED((ACC_N,), jnp.float32),
        pltpu.VMEM((2, CHK, BLK), jnp.int32),
        pltpu.VMEM((2, CHK, BLK), jnp.int32),
        pltpu.VMEM((2, BLK, D_IN), jnp.float32),
        pltpu.VMEM((BLK,), jnp.float32),
        pltpu.VMEM((RPT,), jnp.float32),
        pltpu.SemaphoreType.DMA,
        pltpu.SemaphoreType.DMA,
        pltpu.SemaphoreType.DMA,
    ],
)
def _sc_segsum1(x_hbm, src_hbm, dst_hbm, z2_hbm, z1_hbm, ones_hbm,
                sum_out, deg_out, acc_sh, deg_sh, src_v, dst_v, rows_v,
                ones_v, deg_v, sem, isem, dsem):
    c = lax.axis_index("c")
    s = lax.axis_index("s")
    wid = c * NS + s
    base = wid * NBLK1
    pltpu.sync_copy(ones_hbm, ones_v)
    pltpu.sync_copy(src_hbm.at[pl.ds(base, CHK)], src_v.at[0])
    pltpu.sync_copy(dst_hbm.at[pl.ds(base, CHK)], dst_v.at[0])
    pltpu.sync_copy(z2_hbm.at[pl.ds(s * RPT, RPT)], acc_sh.at[pl.ds(s * RPT, RPT)])
    pltpu.sync_copy(z1_hbm.at[pl.ds(s * RPT, RPT)], deg_v)
    pltpu.sync_copy(deg_v, deg_sh.at[pl.ds(s * RPT, RPT)])
    plsc.subcore_barrier()

    def deg_drain(n):
        for _ in range(n):
            pltpu.make_async_copy(ones_v, deg_sh.at[dst_v.at[0, 0]],
                                  dsem).wait()

    def idx_load(g):
        @pl.when(g >= 2)
        def _():
            deg_drain(CHK)
        pltpu.async_copy(src_hbm.at[pl.ds(base + g * CHK, CHK)],
                         src_v.at[g % 2], isem)
        pltpu.async_copy(dst_hbm.at[pl.ds(base + g * CHK, CHK)],
                         dst_v.at[g % 2], isem)

    def idx_wait():
        pltpu.make_async_copy(src_hbm.at[pl.ds(base, CHK)], src_v.at[0],
                              isem).wait()
        pltpu.make_async_copy(dst_hbm.at[pl.ds(base, CHK)], dst_v.at[0],
                              isem).wait()

    def per_block(par, b):
        pltpu.async_copy(ones_v, deg_sh.at[dst_v.at[par, b]], dsem, add=True)

    _segsum_loop(x_hbm, idx_load, idx_wait, src_v, dst_v, rows_v, acc_sh,
                 sem, NCHK1, per_block)
    deg_drain(2 * CHK)
    plsc.subcore_barrier()
    pltpu.sync_copy(acc_sh.at[pl.ds(s * RPT, RPT)], sum_out.at[c, pl.ds(s * RPT, RPT)])
    pltpu.sync_copy(deg_sh.at[pl.ds(s * RPT, RPT)], deg_v)
    pltpu.sync_copy(deg_v, deg_out.at[pl.ds(c * ACC_N + s * RPT, RPT)])


@functools.partial(
    pl.kernel,
    mesh=_sc_mesh,
    out_type=jax.ShapeDtypeStruct((NC, ACC_N, D_IN), jnp.float32),
    scratch_types=[
        pltpu.VMEM_SHARED((ACC_N, D_IN), jnp.float32),
        pltpu.VMEM((2, CHK, BLK), jnp.int32),
        pltpu.VMEM((2, CHK, BLK), jnp.int32),
        pltpu.VMEM((2, BLK, D_IN), jnp.float32),
        pltpu.SemaphoreType.DMA,
        pltpu.SemaphoreType.DMA,
    ],
)
def _sc_segsum2(h2_hbm, src2_hbm, dst2_hbm, z2_hbm,
                sum_out, acc_sh, src_v, dst_v, rows_v, sem, isem):
    c = lax.axis_index("c")
    s = lax.axis_index("s")
    base = s * NBLK2
    pltpu.sync_copy(src2_hbm.at[c, pl.ds(base, CHK)], src_v.at[0])
    pltpu.sync_copy(dst2_hbm.at[pl.ds(base, CHK)], dst_v.at[0])
    pltpu.sync_copy(z2_hbm.at[pl.ds(s * RPT, RPT)],
                    acc_sh.at[pl.ds(s * RPT, RPT)])
    plsc.subcore_barrier()

    def idx_load(g):
        pltpu.async_copy(src2_hbm.at[c, pl.ds(base + g * CHK, CHK)],
                         src_v.at[g % 2], isem)
        pltpu.async_copy(dst2_hbm.at[pl.ds(base + g * CHK, CHK)],
                         dst_v.at[g % 2], isem)

    def idx_wait():
        pltpu.make_async_copy(src2_hbm.at[c, pl.ds(base, CHK)], src_v.at[0],
                              isem).wait()
        pltpu.make_async_copy(dst2_hbm.at[pl.ds(base, CHK)], dst_v.at[0],
                              isem).wait()

    _segsum_loop(h2_hbm, idx_load, idx_wait, src_v, dst_v, rows_v, acc_sh,
                 sem, NCHK2)
    plsc.subcore_barrier()
    pltpu.sync_copy(acc_sh.at[pl.ds(s * RPT, RPT)],
                    sum_out.at[c, pl.ds(s * RPT, RPT)])


_DN = (((1,), (1,)), ((), ()))


def _dense_xr_body(x_ref, wr_ref, b_ref, out_ref):
    out_ref[...] = (lax.dot_general(
        x_ref[...], wr_ref[...], _DN,
        preferred_element_type=jnp.float32) + b_ref[...]).astype(jnp.bfloat16)


def _dense1_body(parts_ref, degp_ref, xr_ref, wl_ref, h_ref):
    summed = parts_ref[0] + parts_ref[1]
    deg = jnp.maximum(degp_ref[0, :, :1] + degp_ref[1, :, :1], 1.0)
    agg = summed * (1.0 / deg)
    z = lax.dot_general(agg, wl_ref[...], _DN,
                        preferred_element_type=jnp.float32) \
        + xr_ref[...].astype(jnp.float32)
    h = jnp.maximum(z, 0.0)
    h_ref[0] = h[:, :D_IN]
    h_ref[1] = h[:, D_IN:]


def _dense_hr_body(hlo_ref, hhi_ref, wr_ref, b_ref, out_ref):
    out_ref[...] = (
        lax.dot_general(hlo_ref[0], wr_ref[:, :D_IN], _DN,
                        preferred_element_type=jnp.float32)
        + lax.dot_general(hhi_ref[0], wr_ref[:, D_IN:], _DN,
                          preferred_element_type=jnp.float32)
        + b_ref[...]).astype(jnp.bfloat16)


def _dense2_body(parts_ref, degp_ref, hr_ref, wl_ref, out_ref):
    rdeg = 1.0 / jnp.maximum(degp_ref[0, :, :1] + degp_ref[1, :, :1], 1.0)
    out_ref[...] = (
        lax.dot_general(parts_ref[0] * rdeg, wl_ref[:, :D_IN], _DN,
                        preferred_element_type=jnp.float32)
        + lax.dot_general(parts_ref[1] * rdeg, wl_ref[:, D_IN:], _DN,
                          preferred_element_type=jnp.float32)
        + hr_ref[...].astype(jnp.float32))


_BR = 2000
_NRB = N_NODES // _BR


def _full(shape):
    n = len(shape)
    return pl.BlockSpec(shape, lambda *a: (0,) * n)


def kernel(x, edge_index, W1_l, b1, W1_r, W2_l, b2, W2_r):
    eidx = edge_index.astype(jnp.int32)
    srcw, dstw = pl.pallas_call(
        _prep_body,
        grid=(6,),
        in_specs=[pl.BlockSpec((2, _PBR * BLK), lambda i: (0, i))],
        out_specs=[pl.BlockSpec((_PBR, BLK), lambda i: (i, 0)),
                   pl.BlockSpec((_PBR, BLK), lambda i: (i, 0))],
        out_shape=[jax.ShapeDtypeStruct((NB, BLK), jnp.int32)] * 2,
    )(eidx)
    src2w = pl.pallas_call(
        _prep2_body,
        grid=(5,),
        in_specs=[pl.BlockSpec((512, BLK), lambda i: (i, 0))],
        out_specs=pl.BlockSpec((NC, 512, BLK), lambda i: (0, i, 0)),
        out_shape=jax.ShapeDtypeStruct((NC, NB, BLK), jnp.int32),
    )(srcw)

    grid = (_NRB,)
    row_spec = pl.BlockSpec((_BR, D_IN), lambda i: (i, 0))
    hid_spec = pl.BlockSpec((_BR, D_HID), lambda i: (i, 0))
    parts_spec = pl.BlockSpec((NC, _BR, D_IN), lambda i: (0, i, 0))
    deg_spec = pl.BlockSpec((NC, _BR, 8), lambda i: (0, i, 0))

    sum1, deg1 = _sc_segsum1(x, srcw, dstw,
                             jnp.zeros((ACC_N, D_IN), jnp.float32),
                             jnp.zeros((ACC_N,), jnp.float32),
                             jnp.ones((BLK,), jnp.float32))
    xr = pl.pallas_call(
        _dense_xr_body,
        grid=grid,
        in_specs=[row_spec, _full((D_HID, D_IN)), _full((1, D_HID))],
        out_specs=hid_spec,
        out_shape=jax.ShapeDtypeStruct((N_NODES, D_HID), jnp.bfloat16),
    )(x, W1_r, b1.reshape(1, D_HID))

    deg3 = lax.broadcast_in_dim(deg1.reshape(NC, ACC_N), (NC, ACC_N, 8),
                                (0, 1))
    h2 = pl.pallas_call(
        _dense1_body,
        grid=grid,
        in_specs=[parts_spec, deg_spec, hid_spec, _full((D_HID, D_IN))],
        out_specs=pl.BlockSpec((NC, _BR, D_IN), lambda i: (0, i, 0)),
        out_shape=jax.ShapeDtypeStruct((NC, N_NODES, D_IN), jnp.float32),
    )(sum1, deg3, xr, W1_l)

    sum2 = _sc_segsum2(h2.reshape(2 * N_NODES, D_IN), src2w, dstw,
                       jnp.zeros((ACC_N, D_IN), jnp.float32))
    hr = pl.pallas_call(
        _dense_hr_body,
        grid=grid,
        in_specs=[
            pl.BlockSpec((1, _BR, D_IN), lambda i: (0, i, 0)),
            pl.BlockSpec((1, _BR, D_IN), lambda i: (1, i, 0)),
            _full((D_OUT, D_HID)),
            _full((1, D_OUT)),
        ],
        out_specs=pl.BlockSpec((_BR, D_OUT), lambda i: (i, 0)),
        out_shape=jax.ShapeDtypeStruct((N_NODES, D_OUT), jnp.bfloat16),
    )(h2, h2, W2_r, b2.reshape(1, D_OUT))

    out = pl.pallas_call(
        _dense2_body,
        grid=grid,
        in_specs=[parts_spec, deg_spec,
                  pl.BlockSpec((_BR, D_OUT), lambda i: (i, 0)),
                  _full((D_OUT, D_HID))],
        out_specs=pl.BlockSpec((_BR, D_OUT), lambda i: (i, 0)),
        out_shape=jax.ShapeDtypeStruct((N_NODES, D_OUT), jnp.float32),
    )(sum2, deg3, hr, W2_l)
    return out
